# baseline (speedup 1.0000x reference)
import jax
import jax.numpy as jnp
from jax.experimental import pallas as pl
from jax.experimental.pallas import tpu as pltpu

_BLOCK_COLS = 65536


def _body(srow_ref, scol_ref, ct_ref, ind_ref, t_ref, out_ref, indout_ref):
    j = pl.program_id(0)
    row = srow_ref[0]
    col = scol_ref[0]
    base = j * _BLOCK_COLS
    blk = ct_ref[...]
    hit = jnp.logical_and(col >= base, col < base + _BLOCK_COLS)

    @pl.when(hit)
    def _():
        rows = jax.lax.broadcasted_iota(jnp.int32, blk.shape, 0)
        cols = jax.lax.broadcasted_iota(jnp.int32, blk.shape, 1) + base
        mask = jnp.logical_and(rows == row, cols == col)
        out_ref[...] = jnp.where(mask, t_ref[0, 0], blk)

    @pl.when(jnp.logical_not(hit))
    def _():
        out_ref[...] = blk

    @pl.when(j == 0)
    def _():
        lanes = jax.lax.broadcasted_iota(jnp.int32, ind_ref.shape, 1)
        indout_ref[...] = ind_ref[...] + (lanes == row).astype(jnp.int32)


def kernel(clicktimes, indices, idx, t):
    n_det, n_cols = clicktimes.shape
    grid = n_cols // _BLOCK_COLS
    row = jnp.asarray(idx, jnp.int32).reshape(1)
    col = jnp.take(indices, jnp.asarray(idx, jnp.int32)).reshape(1)
    ind2d = indices.reshape(1, n_det)
    t2d = jnp.asarray(t, jnp.float32).reshape(1, 1)

    out, indout = pl.pallas_call(
        _body,
        grid_spec=pltpu.PrefetchScalarGridSpec(
            num_scalar_prefetch=2,
            grid=(grid,),
            in_specs=[
                pl.BlockSpec((n_det, _BLOCK_COLS), lambda j, s1, s2: (0, j)),
                pl.BlockSpec((1, n_det), lambda j, s1, s2: (0, 0)),
                pl.BlockSpec((1, 1), lambda j, s1, s2: (0, 0)),
            ],
            out_specs=[
                pl.BlockSpec((n_det, _BLOCK_COLS), lambda j, s1, s2: (0, j)),
                pl.BlockSpec((1, n_det), lambda j, s1, s2: (0, 0)),
            ],
        ),
        out_shape=[
            jax.ShapeDtypeStruct((n_det, n_cols), clicktimes.dtype),
            jax.ShapeDtypeStruct((1, n_det), indices.dtype),
        ],
    )(row, col, clicktimes, ind2d, t2d)
    return (out, indout.reshape(n_det))


# speedup vs baseline: 1.2087x; 1.2087x over previous
import jax
import jax.numpy as jnp
from jax.experimental import pallas as pl
from jax.experimental.pallas import tpu as pltpu

_BLOCK_ROWS = 32


def _body(srow_ref, scol_ref, ct_ref, ind_ref, t_ref, out_ref, indout_ref):
    j = pl.program_id(0)
    row = srow_ref[0]
    col = scol_ref[0]
    base = j * _BLOCK_ROWS
    blk = ct_ref[...]
    hit = jnp.logical_and(row >= base, row < base + _BLOCK_ROWS)

    @pl.when(hit)
    def _():
        rows = jax.lax.broadcasted_iota(jnp.int32, blk.shape, 0) + base
        cols = jax.lax.broadcasted_iota(jnp.int32, blk.shape, 1)
        mask = jnp.logical_and(rows == row, cols == col)
        out_ref[...] = jnp.where(mask, t_ref[0, 0], blk)

    @pl.when(jnp.logical_not(hit))
    def _():
        out_ref[...] = blk

    @pl.when(j == 0)
    def _():
        lanes = jax.lax.broadcasted_iota(jnp.int32, ind_ref.shape, 1)
        indout_ref[...] = ind_ref[...] + (lanes == row).astype(jnp.int32)


def kernel(clicktimes, indices, idx, t):
    n_det, n_cols = clicktimes.shape
    grid = n_det // _BLOCK_ROWS
    row = jnp.asarray(idx, jnp.int32).reshape(1)
    col = jnp.take(indices, jnp.asarray(idx, jnp.int32)).reshape(1)
    ind2d = indices.reshape(1, n_det)
    t2d = jnp.asarray(t, jnp.float32).reshape(1, 1)

    out, indout = pl.pallas_call(
        _body,
        grid_spec=pltpu.PrefetchScalarGridSpec(
            num_scalar_prefetch=2,
            grid=(grid,),
            in_specs=[
                pl.BlockSpec((_BLOCK_ROWS, n_cols), lambda j, s1, s2: (j, 0)),
                pl.BlockSpec((1, n_det), lambda j, s1, s2: (0, 0)),
                pl.BlockSpec((1, 1), lambda j, s1, s2: (0, 0)),
            ],
            out_specs=[
                pl.BlockSpec((_BLOCK_ROWS, n_cols), lambda j, s1, s2: (j, 0)),
                pl.BlockSpec((1, n_det), lambda j, s1, s2: (0, 0)),
            ],
        ),
        out_shape=[
            jax.ShapeDtypeStruct((n_det, n_cols), clicktimes.dtype),
            jax.ShapeDtypeStruct((1, n_det), indices.dtype),
        ],
    )(row, col, clicktimes, ind2d, t2d)
    return (out, indout.reshape(n_det))
